# baseline (speedup 1.0000x reference)
import functools

import jax
import jax.numpy as jnp
from jax.experimental import pallas as pl


def _level_body(p_ref, gt_ref, cf_ref, part_ref, o_ref):
    j = pl.program_id(0)
    x = p_ref[...]
    s = jax.nn.sigmoid(x[:, :, :6])
    cf = cf_ref[...]
    offx = cf[:, 0][None, :]
    offy = cf[:, 1][None, :]
    cw2 = cf[:, 2][None, :]
    ch2 = cf[:, 3][None, :]
    s2 = cf[:, 4][None, :]
    th = cf[:, 5][None, :]

    xc = s[:, :, 0] * s2 + offx
    yc = s[:, :, 1] * s2 + offy
    hw = s[:, :, 2] * s[:, :, 2] * cw2
    hh = s[:, :, 3] * s[:, :, 3] * ch2
    x1 = xc - hw
    x2 = xc + hw
    y1 = yc - hh
    y2 = yc + hh

    g = gt_ref[...]
    ix1 = jnp.maximum(x1, g[:, 0:1])
    iy1 = jnp.maximum(y1, g[:, 1:2])
    ix2 = jnp.minimum(x2, g[:, 2:3])
    iy2 = jnp.minimum(y2, g[:, 3:4])
    inter = jnp.maximum(ix2 - ix1, 0.0) * jnp.maximum(iy2 - iy1, 0.0)
    area_b = (hw + hw) * (hh + hh)
    area_g = (g[:, 2:3] - g[:, 0:1]) * (g[:, 3:4] - g[:, 1:2])
    iou = inter / (area_b + area_g - inter)

    score = s[:, :, 4] * s[:, :, 5]
    val = jnp.where(iou >= th, score, 0.0)
    local = jnp.max(val, axis=1)
    cur = jnp.broadcast_to(local[:, None], o_ref.shape)

    @pl.when(j == 0)
    def _():
        o_ref[...] = jnp.maximum(part_ref[...], cur)

    @pl.when(j != 0)
    def _():
        o_ref[...] = jnp.maximum(o_ref[...], cur)


def _run_level(p, gt, coef, partial, c_block):
    bs, n, ch = p.shape
    grid = n // c_block
    return pl.pallas_call(
        _level_body,
        grid=(grid,),
        in_specs=[
            pl.BlockSpec((bs, c_block, ch), lambda j: (0, j, 0)),
            pl.BlockSpec((bs, 4), lambda j: (0, 0)),
            pl.BlockSpec((c_block, 8), lambda j: (j, 0)),
            pl.BlockSpec((bs, 128), lambda j: (0, 0)),
        ],
        out_specs=pl.BlockSpec((bs, 128), lambda j: (0, 0)),
        out_shape=jax.ShapeDtypeStruct((bs, 128), jnp.float32),
    )(p, gt, coef, partial)


def _coef_table(nx, ny, na, stride, anchors_i, iou_thresh):
    n = na * ny * nx
    idx = jnp.arange(n, dtype=jnp.int32)
    gx = (idx % nx).astype(jnp.float32)
    gy = ((idx // nx) % ny).astype(jnp.float32)
    a = idx // (nx * ny)
    aw = anchors_i[:, 0][a]
    ah = anchors_i[:, 1][a]
    z = jnp.zeros((n,), jnp.float32)
    return jnp.stack([
        (gx - 0.5) * stride,
        (gy - 0.5) * stride,
        2.0 * aw * stride,
        2.0 * ah * stride,
        jnp.broadcast_to(2.0 * stride, (n,)),
        jnp.broadcast_to(jnp.asarray(iou_thresh, jnp.float32), (n,)),
        z, z,
    ], axis=1)


def kernel(p0, p1, p2, gt, anchors, strides, iou_thresh):
    bs = p0.shape[0]
    partial = jnp.zeros((bs, 128), jnp.float32)
    blocks = (1600, 1600, 1200)
    for i, p in enumerate((p0, p1, p2)):
        _, na, ny, nx, ch = p.shape
        pr = p.reshape(bs, na * ny * nx, ch)
        coef = _coef_table(nx, ny, na, strides[i], anchors[i], iou_thresh)
        partial = _run_level(pr, gt, coef, partial, blocks[i])
    det = partial[:, 0]
    return det.mean(), det


# speedup vs baseline: 1.6133x; 1.6133x over previous
import jax
import jax.numpy as jnp
from jax import lax
from jax.experimental import pallas as pl
from jax.experimental.pallas import tpu as pltpu
from jax.experimental.pallas import tpu_sc as plsc

_NW = 32
_WPI = 4
_LANES = 16
_CH = 85

_LEVELS = (
    (19200, 80, 80, 400, 34000),
    (4800, 40, 40, 400, 34000),
    (1200, 20, 20, 300, 25504),
)


def _sc_body(p0_hbm, p1_hbm, p2_hbm, gt_hbm, lvl_hbm, out_hbm,
             buf_v, gt_v, lvl_v, acc_v):
    w = lax.axis_index("s") * 2 + lax.axis_index("c")
    b = w // _WPI
    q = w % _WPI

    pltpu.sync_copy(gt_hbm.at[b], gt_v)
    pltpu.sync_copy(lvl_hbm, lvl_v)

    g = gt_v[...]
    gx1 = g[0]
    gy1 = g[1]
    gx2 = g[2]
    gy2 = g[3]
    area_g = g[4]
    th = g[5]

    acc_v[...] = jnp.zeros((_LANES,), jnp.float32)
    lane = lax.iota(jnp.int32, _LANES)

    for i, (ncl, nx, ny, kc, kw) in enumerate(_LEVELS):
        p_hbm = (p0_hbm, p1_hbm, p2_hbm)[i]
        span = ncl // _WPI
        nchunks = span // kc
        ngroups = (kc + _LANES - 1) // _LANES
        tail = kc % _LANES != 0
        lv = lvl_v[i, :]
        s = lv[0]
        aw = (lv[1], lv[3], lv[5])
        ah = (lv[2], lv[4], lv[6])
        c0w = q * span

        def chunk_body(j, _, p_hbm=p_hbm, kc=kc, kw=kw, ngroups=ngroups,
                       tail=tail, nx=nx, ny=ny, s=s, aw=aw, ah=ah, c0w=c0w,
                       ncl=ncl):
            cst = c0w + j * kc
            word0 = (b * ncl + cst) * _CH
            start8 = pl.multiple_of(word0 & ~7, 8)
            delta = word0 - start8
            pltpu.sync_copy(p_hbm.at[pl.ds(start8, kw)],
                            buf_v.at[pl.ds(0, kw)])

            def group_body(gi, _):
                r = gi * _LANES + lane
                if tail:
                    valid = r < kc
                    r = jnp.minimum(r, kc - 1)
                base = delta + r * _CH
                ch = [plsc.load_gather(buf_v, [base + k]) for k in range(6)]
                sx, sy, sw, sh, so, sc = [1.0 / (1.0 + jnp.exp(-c))
                                          for c in ch]

                cf = (cst + r).astype(jnp.float32)
                q1 = (cf * (1.0 / nx)).astype(jnp.int32)
                a = (q1.astype(jnp.float32) * (1.0 / ny)).astype(jnp.int32)
                gx = cf - q1.astype(jnp.float32) * float(nx)
                gy = (q1 - a * ny).astype(jnp.float32)

                a1 = a == 1
                a2 = a == 2
                aws = jnp.where(a2, aw[2], jnp.where(a1, aw[1], aw[0]))
                ahs = jnp.where(a2, ah[2], jnp.where(a1, ah[1], ah[0]))

                xc = (sx * 2.0 - 0.5 + gx) * s
                yc = (sy * 2.0 - 0.5 + gy) * s
                hw = sw * sw * (2.0 * aws)
                hh = sh * sh * (2.0 * ahs)

                ix1 = jnp.maximum(xc - hw, gx1)
                iy1 = jnp.maximum(yc - hh, gy1)
                ix2 = jnp.minimum(xc + hw, gx2)
                iy2 = jnp.minimum(yc + hh, gy2)
                inter = (jnp.maximum(ix2 - ix1, 0.0)
                         * jnp.maximum(iy2 - iy1, 0.0))
                area_b = (hw + hw) * (hh + hh)
                iou = inter / (area_b + area_g - inter)

                keep = iou >= th
                if tail:
                    keep = jnp.logical_and(keep, valid)
                val = jnp.where(keep, so * sc, 0.0)
                acc_v[...] = jnp.maximum(acc_v[...], val)
                return 0

            lax.fori_loop(0, ngroups, group_body, 0)
            return 0

        lax.fori_loop(0, nchunks, chunk_body, 0)

    pltpu.sync_copy(acc_v, out_hbm.at[w])


def kernel(p0, p1, p2, gt, anchors, strides, iou_thresh):
    bs = p0.shape[0]
    ps = [p.reshape(-1) for p in (p0, p1, p2)]

    area_g = (gt[:, 2] - gt[:, 0]) * (gt[:, 3] - gt[:, 1])
    gt_pad = jnp.concatenate([
        gt, area_g[:, None],
        jnp.broadcast_to(jnp.asarray(iou_thresh, jnp.float32), (bs, 1)),
        jnp.zeros((bs, 10), jnp.float32)], axis=1)

    anchor_s = (anchors * strides[:, None, None]).reshape(3, 6)
    lvl = jnp.concatenate([strides[:, None], anchor_s,
                           jnp.zeros((3, 9), jnp.float32)], axis=1)

    mesh = plsc.VectorSubcoreMesh(core_axis_name="c", subcore_axis_name="s")
    run = pl.kernel(
        _sc_body, mesh=mesh,
        compiler_params=pltpu.CompilerParams(needs_layout_passes=False),
        out_type=jax.ShapeDtypeStruct((_NW, _LANES), jnp.float32),
        scratch_types=[
            pltpu.VMEM((34000,), jnp.float32),
            pltpu.VMEM((_LANES,), jnp.float32),
            pltpu.VMEM((3, _LANES), jnp.float32),
            pltpu.VMEM((_LANES,), jnp.float32),
        ],
    )
    part = run(ps[0], ps[1], ps[2], gt_pad, lvl)

    det = jnp.max(part.reshape(bs, _WPI * _LANES), axis=1)
    return det.mean(), det
